# baseline (speedup 1.0000x reference)
import jax
import jax.numpy as jnp
from jax.experimental import pallas as pl
from jax.experimental.pallas import tpu as pltpu

B, N, NFEAT, NHID1, NHID2 = 8, 2048, 128, 64, 32
ROW_TILE = 256


def _gcn_kernel(length_ref, x_ref, adj_ref, W1_ref, b1_ref, W2_ref, b2_ref,
                Wlin_ref, blin_ref, out_ref, s2_scr):
    b = pl.program_id(0)
    xb = x_ref[0]

    s1 = jnp.dot(xb, W1_ref[:], preferred_element_type=jnp.float32)
    for t in range(N // ROW_TILE):
        ht = jnp.dot(adj_ref[0, t * ROW_TILE:(t + 1) * ROW_TILE, :], s1,
                     preferred_element_type=jnp.float32) + b1_ref[:]
        ht = jnp.maximum(ht, 0.0)
        s2_scr[t * ROW_TILE:(t + 1) * ROW_TILE, :] = jnp.dot(
            ht, W2_ref[:], preferred_element_type=jnp.float32)
    s2 = s2_scr[:]

    L = length_ref[b]
    n_tiles = (L + ROW_TILE - 1) // ROW_TILE
    tile_iota = jax.lax.broadcasted_iota(jnp.int32, (ROW_TILE, 1), 0)

    def tile_body(t, acc):
        r0 = t * ROW_TILE
        z = jnp.dot(adj_ref[0, pl.ds(r0, ROW_TILE), :], s2,
                    preferred_element_type=jnp.float32) + b2_ref[:]
        z = jnp.maximum(z, 0.0)
        z = jnp.where(tile_iota + r0 < L, z, 0.0)
        return acc + jnp.sum(z, axis=0, keepdims=True)

    pooled = jax.lax.fori_loop(
        0, n_tiles, tile_body, jnp.zeros((1, NHID2), jnp.float32))
    pooled = pooled / L.astype(jnp.float32)

    out_ref[pl.ds(b, 1), :] = jnp.dot(
        pooled, Wlin_ref[:], preferred_element_type=jnp.float32) + blin_ref[:]


def kernel(x, adj, length, W1, b1, W2, b2, Wlin, blin):
    b1r = b1.reshape(1, NHID1)
    b2r = b2.reshape(1, NHID2)
    blinr = blin.reshape(1, 1)

    grid_spec = pltpu.PrefetchScalarGridSpec(
        num_scalar_prefetch=1,
        grid=(B,),
        in_specs=[
            pl.BlockSpec((1, N, NFEAT), lambda b, L: (b, 0, 0)),
            pl.BlockSpec((1, N, N), lambda b, L: (b, 0, 0)),
            pl.BlockSpec((NFEAT, NHID1), lambda b, L: (0, 0)),
            pl.BlockSpec((1, NHID1), lambda b, L: (0, 0)),
            pl.BlockSpec((NHID1, NHID2), lambda b, L: (0, 0)),
            pl.BlockSpec((1, NHID2), lambda b, L: (0, 0)),
            pl.BlockSpec((NHID2, 1), lambda b, L: (0, 0)),
            pl.BlockSpec((1, 1), lambda b, L: (0, 0)),
        ],
        out_specs=pl.BlockSpec((B, 1), lambda b, L: (0, 0)),
        scratch_shapes=[pltpu.VMEM((N, NHID2), jnp.float32)],
    )

    out = pl.pallas_call(
        _gcn_kernel,
        grid_spec=grid_spec,
        out_shape=jax.ShapeDtypeStruct((B, 1), jnp.float32),
    )(length, x, adj, W1, b1r, W2, b2r, Wlin, blinr)
    return out


# speedup vs baseline: 1.2445x; 1.2445x over previous
import jax
import jax.numpy as jnp
from jax.experimental import pallas as pl
from jax.experimental.pallas import tpu as pltpu

B, N, NFEAT, NHID1, NHID2 = 8, 2048, 128, 64, 32
ROW_TILE = 256


def _gcn_kernel(length_ref, x_ref, adj_ref, W1_ref, b1_ref, W2_ref, b2_ref,
                Wlin_ref, blin_ref, out_ref, s2_scr):
    b = pl.program_id(0)
    xb = x_ref[0]

    s1 = jnp.dot(xb, W1_ref[:], preferred_element_type=jnp.float32)
    h = jnp.dot(adj_ref[0], s1, preferred_element_type=jnp.float32) + b1_ref[:]
    h = jnp.maximum(h, 0.0)
    s2 = jnp.dot(h, W2_ref[:], preferred_element_type=jnp.float32)
    del s2_scr

    L = length_ref[b]
    n_tiles = (L + ROW_TILE - 1) // ROW_TILE
    tile_iota = jax.lax.broadcasted_iota(jnp.int32, (ROW_TILE, 1), 0)

    def tile_body(t, acc):
        r0 = t * ROW_TILE
        z = jnp.dot(adj_ref[0, pl.ds(r0, ROW_TILE), :], s2,
                    preferred_element_type=jnp.float32) + b2_ref[:]
        z = jnp.maximum(z, 0.0)
        z = jnp.where(tile_iota + r0 < L, z, 0.0)
        return acc + jnp.sum(z, axis=0, keepdims=True)

    pooled = jax.lax.fori_loop(
        0, n_tiles, tile_body, jnp.zeros((1, NHID2), jnp.float32))
    pooled = pooled / L.astype(jnp.float32)

    out_ref[pl.ds(b, 1), :] = jnp.dot(
        pooled, Wlin_ref[:], preferred_element_type=jnp.float32) + blin_ref[:]


def kernel(x, adj, length, W1, b1, W2, b2, Wlin, blin):
    b1r = b1.reshape(1, NHID1)
    b2r = b2.reshape(1, NHID2)
    blinr = blin.reshape(1, 1)

    grid_spec = pltpu.PrefetchScalarGridSpec(
        num_scalar_prefetch=1,
        grid=(B,),
        in_specs=[
            pl.BlockSpec((1, N, NFEAT), lambda b, L: (b, 0, 0)),
            pl.BlockSpec((1, N, N), lambda b, L: (b, 0, 0)),
            pl.BlockSpec((NFEAT, NHID1), lambda b, L: (0, 0)),
            pl.BlockSpec((1, NHID1), lambda b, L: (0, 0)),
            pl.BlockSpec((NHID1, NHID2), lambda b, L: (0, 0)),
            pl.BlockSpec((1, NHID2), lambda b, L: (0, 0)),
            pl.BlockSpec((NHID2, 1), lambda b, L: (0, 0)),
            pl.BlockSpec((1, 1), lambda b, L: (0, 0)),
        ],
        out_specs=pl.BlockSpec((B, 1), lambda b, L: (0, 0)),
        scratch_shapes=[pltpu.VMEM((N, NHID2), jnp.float32)],
    )

    out = pl.pallas_call(
        _gcn_kernel,
        grid_spec=grid_spec,
        out_shape=jax.ShapeDtypeStruct((B, 1), jnp.float32),
    )(length, x, adj, W1, b1r, W2, b2r, Wlin, blinr)
    return out


# speedup vs baseline: 1.2765x; 1.0258x over previous
import jax
import jax.numpy as jnp
from jax.experimental import pallas as pl
from jax.experimental.pallas import tpu as pltpu

B, N, NFEAT, NHID1, NHID2 = 8, 2048, 128, 64, 32
ROW_TILE = 512


def _gcn_kernel(length_ref, x_ref, adj_ref, W1_ref, b1_ref, W2_ref, b2_ref,
                Wlin_ref, blin_ref, out_ref, s2_scr):
    b = pl.program_id(0)
    xb = x_ref[0]

    s1 = jnp.dot(xb, W1_ref[:], preferred_element_type=jnp.float32)
    h = jnp.dot(adj_ref[0], s1, preferred_element_type=jnp.float32) + b1_ref[:]
    h = jnp.maximum(h, 0.0)
    s2 = jnp.dot(h, W2_ref[:], preferred_element_type=jnp.float32)
    del s2_scr

    L = length_ref[b]
    n_tiles = (L + ROW_TILE - 1) // ROW_TILE
    tile_iota = jax.lax.broadcasted_iota(jnp.int32, (ROW_TILE, 1), 0)

    def tile_body(t, acc):
        r0 = t * ROW_TILE
        z = jnp.dot(adj_ref[0, pl.ds(r0, ROW_TILE), :], s2,
                    preferred_element_type=jnp.float32) + b2_ref[:]
        z = jnp.maximum(z, 0.0)
        z = jnp.where(tile_iota + r0 < L, z, 0.0)
        return acc + jnp.sum(z, axis=0, keepdims=True)

    pooled = jax.lax.fori_loop(
        0, n_tiles, tile_body, jnp.zeros((1, NHID2), jnp.float32))
    pooled = pooled / L.astype(jnp.float32)

    out_ref[pl.ds(b, 1), :] = jnp.dot(
        pooled, Wlin_ref[:], preferred_element_type=jnp.float32) + blin_ref[:]


def kernel(x, adj, length, W1, b1, W2, b2, Wlin, blin):
    b1r = b1.reshape(1, NHID1)
    b2r = b2.reshape(1, NHID2)
    blinr = blin.reshape(1, 1)

    grid_spec = pltpu.PrefetchScalarGridSpec(
        num_scalar_prefetch=1,
        grid=(B,),
        in_specs=[
            pl.BlockSpec((1, N, NFEAT), lambda b, L: (b, 0, 0)),
            pl.BlockSpec((1, N, N), lambda b, L: (b, 0, 0)),
            pl.BlockSpec((NFEAT, NHID1), lambda b, L: (0, 0)),
            pl.BlockSpec((1, NHID1), lambda b, L: (0, 0)),
            pl.BlockSpec((NHID1, NHID2), lambda b, L: (0, 0)),
            pl.BlockSpec((1, NHID2), lambda b, L: (0, 0)),
            pl.BlockSpec((NHID2, 1), lambda b, L: (0, 0)),
            pl.BlockSpec((1, 1), lambda b, L: (0, 0)),
        ],
        out_specs=pl.BlockSpec((B, 1), lambda b, L: (0, 0)),
        scratch_shapes=[pltpu.VMEM((N, NHID2), jnp.float32)],
    )

    out = pl.pallas_call(
        _gcn_kernel,
        grid_spec=grid_spec,
        out_shape=jax.ShapeDtypeStruct((B, 1), jnp.float32),
    )(length, x, adj, W1, b1r, W2, b2r, Wlin, blinr)
    return out


# speedup vs baseline: 1.2818x; 1.0041x over previous
import jax
import jax.numpy as jnp
from jax.experimental import pallas as pl
from jax.experimental.pallas import tpu as pltpu

B, N, NFEAT, NHID1, NHID2 = 8, 2048, 128, 64, 32
ROW_TILE = 1024


def _gcn_kernel(length_ref, x_ref, adj_ref, W1_ref, b1_ref, W2_ref, b2_ref,
                Wlin_ref, blin_ref, out_ref, s2_scr):
    b = pl.program_id(0)
    xb = x_ref[0]

    s1 = jnp.dot(xb, W1_ref[:], preferred_element_type=jnp.float32)
    h = jnp.dot(adj_ref[0], s1, preferred_element_type=jnp.float32) + b1_ref[:]
    h = jnp.maximum(h, 0.0)
    s2 = jnp.dot(h, W2_ref[:], preferred_element_type=jnp.float32)
    del s2_scr

    L = length_ref[b]
    n_tiles = (L + ROW_TILE - 1) // ROW_TILE
    tile_iota = jax.lax.broadcasted_iota(jnp.int32, (ROW_TILE, 1), 0)

    def tile_body(t, acc):
        r0 = t * ROW_TILE
        z = jnp.dot(adj_ref[0, pl.ds(r0, ROW_TILE), :], s2,
                    preferred_element_type=jnp.float32) + b2_ref[:]
        z = jnp.maximum(z, 0.0)
        z = jnp.where(tile_iota + r0 < L, z, 0.0)
        return acc + jnp.sum(z, axis=0, keepdims=True)

    pooled = jax.lax.fori_loop(
        0, n_tiles, tile_body, jnp.zeros((1, NHID2), jnp.float32))
    pooled = pooled / L.astype(jnp.float32)

    out_ref[pl.ds(b, 1), :] = jnp.dot(
        pooled, Wlin_ref[:], preferred_element_type=jnp.float32) + blin_ref[:]


def kernel(x, adj, length, W1, b1, W2, b2, Wlin, blin):
    b1r = b1.reshape(1, NHID1)
    b2r = b2.reshape(1, NHID2)
    blinr = blin.reshape(1, 1)

    grid_spec = pltpu.PrefetchScalarGridSpec(
        num_scalar_prefetch=1,
        grid=(B,),
        in_specs=[
            pl.BlockSpec((1, N, NFEAT), lambda b, L: (b, 0, 0)),
            pl.BlockSpec((1, N, N), lambda b, L: (b, 0, 0)),
            pl.BlockSpec((NFEAT, NHID1), lambda b, L: (0, 0)),
            pl.BlockSpec((1, NHID1), lambda b, L: (0, 0)),
            pl.BlockSpec((NHID1, NHID2), lambda b, L: (0, 0)),
            pl.BlockSpec((1, NHID2), lambda b, L: (0, 0)),
            pl.BlockSpec((NHID2, 1), lambda b, L: (0, 0)),
            pl.BlockSpec((1, 1), lambda b, L: (0, 0)),
        ],
        out_specs=pl.BlockSpec((B, 1), lambda b, L: (0, 0)),
        scratch_shapes=[pltpu.VMEM((N, NHID2), jnp.float32)],
    )

    out = pl.pallas_call(
        _gcn_kernel,
        grid_spec=grid_spec,
        out_shape=jax.ShapeDtypeStruct((B, 1), jnp.float32),
    )(length, x, adj, W1, b1r, W2, b2r, Wlin, blinr)
    return out
